# baseline (speedup 1.0000x reference)
import functools

import jax
import jax.numpy as jnp
from jax import lax
from jax.experimental import pallas as pl
from jax.experimental.pallas import tpu as pltpu
from jax.experimental.pallas import tpu_sc as plsc

L = 16


def _iota():
    return lax.broadcasted_iota(jnp.int32, (L,), 0)


@functools.cache
def _build_transpose(NCELL, C):
    info = plsc.get_sparse_core_info()
    NC, NS = info.num_cores, info.num_subcores
    NW = NC * NS
    CW = NCELL // NW
    BLK = 768
    NB = CW // BLK
    P = BLK + 1
    assert NCELL % NW == 0 and CW % (2 * BLK) == 0 and C == 2 * L

    mesh = plsc.VectorSubcoreMesh(core_axis_name="c", subcore_axis_name="s")

    @functools.partial(
        pl.kernel,
        out_type=jax.ShapeDtypeStruct((NCELL, C // 2), jnp.int32),
        mesh=mesh,
        compiler_params=pltpu.CompilerParams(needs_layout_passes=False,
                                             use_tc_tiling_on_sc=False),
        scratch_types=[
            pltpu.VMEM((2, C, P), jnp.float32),
            pltpu.VMEM((2, BLK, C // 2), jnp.int32),
            pltpu.SemaphoreType.DMA,
            pltpu.SemaphoreType.DMA,
        ],
    )
    def transpose_sc(vox_hbm, tab_hbm, stage_v, ob_v, sem_i, sem_o):
        wid = lax.axis_index("s") * NC + lax.axis_index("c")
        wbase = wid * CW
        lane = _iota()

        def fire_stage(b, par):
            pltpu.async_copy(vox_hbm.at[:, pl.ds(wbase + b * BLK, BLK)],
                             stage_v.at[par, :, pl.ds(0, BLK)], sem_i)

        def wait_stage(b, par):
            pltpu.make_async_copy(vox_hbm.at[:, pl.ds(wbase + b * BLK, BLK)],
                                  stage_v.at[par, :, pl.ds(0, BLK)],
                                  sem_i).wait()

        def fire_out(b, par):
            pltpu.async_copy(ob_v.at[par],
                             tab_hbm.at[pl.ds(wbase + b * BLK, BLK)], sem_o)

        def drain_out(b, par):
            pltpu.make_async_copy(ob_v.at[par],
                                  tab_hbm.at[pl.ds(wbase + b * BLK, BLK)],
                                  sem_o).wait()

        fire_stage(0, 0)
        a_lo = lane * P
        a_hi = (lane + L) * P

        def body(i, carry):
            for par in (0, 1):
                b = i * 2 + par
                nxt = 1 - par

                @pl.when(b + 1 < NB)
                def _prefetch():
                    fire_stage(b + 1, nxt)

                wait_stage(b, par)

                @pl.when(b >= 2)
                def _drain():
                    drain_out(b - 2, par)

                sp = stage_v.at[par]
                op = ob_v.at[par]

                def jbody(j4, carry2):
                    rnd = jnp.full((L,), 0x8000, jnp.int32)
                    msk = jnp.full((L,), -0x10000, jnp.int32)
                    for u in range(4):
                        j = j4 * 4 + u
                        jv = jnp.full((L,), j, jnp.int32)
                        v0 = plsc.load_gather(sp, [lane, jv])
                        v1 = plsc.load_gather(sp, [lane + L, jv])
                        w0 = lax.shift_right_logical(
                            plsc.bitcast(v0, jnp.int32) + rnd, 16)
                        w1 = (plsc.bitcast(v1, jnp.int32) + rnd) & msk
                        plsc.store_scatter(op, [jv, lane], w0 | w1)
                    return carry2

                lax.fori_loop(0, BLK // 4, jbody, 0)
                fire_out(b, par)
            return carry

        lax.fori_loop(0, NB // 2, body, 0)
        drain_out(NB - 2, 0)
        drain_out(NB - 1, 1)

    return transpose_sc


@functools.cache
def _build_sc_call(M, C, D, H, W):
    info = plsc.get_sparse_core_info()
    NC, NS = info.num_cores, info.num_subcores
    NW = NC * NS
    PW = M // NW
    NG = PW // L
    CH = C // L
    assert M % NW == 0 and PW % (2 * L) == 0 and C % L == 0

    mesh = plsc.VectorSubcoreMesh(core_axis_name="c", subcore_axis_name="s")

    @functools.partial(
        pl.kernel,
        out_type=jax.ShapeDtypeStruct((M, C), jnp.float32),
        mesh=mesh,
        compiler_params=pltpu.CompilerParams(needs_layout_passes=False,
                                             use_tc_tiling_on_sc=False),
        scratch_types=[
            pltpu.VMEM((PW * 3,), jnp.float32),
            pltpu.VMEM((2, 8 * L), jnp.int32),
            pltpu.VMEM((2, 8 * L, C // 2), jnp.int32),
            pltpu.VMEM((2, 8, L), jnp.float32),
            pltpu.VMEM((2, L, C), jnp.float32),
            pltpu.VMEM((8, L), jnp.float32),
            pltpu.SemaphoreType.DMA,
            pltpu.SemaphoreType.DMA,
        ],
    )
    def grid_sample_sc(xc_hbm, table_hbm, par_hbm, out_hbm,
                       coords_v, idx_v, rows_v, w_v, out_v, par_v,
                       sem_g, sem_o):
        wid = lax.axis_index("s") * NC + lax.axis_index("c")
        wbase = wid * PW
        pltpu.sync_copy(par_hbm, par_v)
        pltpu.sync_copy(xc_hbm.at[pl.ds(wbase * 3, PW * 3)], coords_v)
        lane = _iota()

        def a_phase(g, par):
            a3 = (g * L + lane) * 3
            gx = plsc.load_gather(coords_v, [a3])
            gy = plsc.load_gather(coords_v, [a3 + 1])
            gz = plsc.load_gather(coords_v, [a3 + 2])

            def axis(gc, arow, brow, size):
                i = jnp.minimum(jnp.maximum(par_v[arow] * gc + par_v[brow],
                                            0.0), float(size - 1))
                i0 = i.astype(jnp.int32)
                w1 = i - i0.astype(jnp.float32)
                i1 = jnp.minimum(i0 + 1, size - 1)
                return i0, i1, w1

            x0, x1, wx = axis(gx, 0, 1, W)
            y0, y1, wy = axis(gy, 2, 3, H)
            z0, z1, wz = axis(gz, 4, 5, D)

            zy = [z0 * (H * W) + y0 * W, z0 * (H * W) + y1 * W,
                  z1 * (H * W) + y0 * W, z1 * (H * W) + y1 * W]
            wzy = [(1.0 - wz) * (1.0 - wy), (1.0 - wz) * wy,
                   wz * (1.0 - wy), wz * wy]
            for k in range(8):
                r = zy[k >> 1] + (x0 if k & 1 == 0 else x1)
                wk = wzy[k >> 1] * ((1.0 - wx) if k & 1 == 0 else wx)
                idx_v[par, pl.ds(k * L, L)] = r
                w_v[par, k] = wk

        def fire_gather(par):
            pltpu.async_copy(table_hbm.at[idx_v.at[par]], rows_v.at[par],
                             sem_g)

        def wait_gather(par):
            pltpu.make_async_copy(table_hbm.at[idx_v.at[par]],
                                  rows_v.at[par], sem_g).wait()

        def b_phase(g, par):
            wv = [w_v[par, k] for k in range(8)]
            for p in range(L):
                wb = [jnp.full((L,), wv[k][p], jnp.float32)
                      for k in range(8)]
                msk = jnp.full((L,), -0x10000, jnp.int32)
                lo, hi = [], []
                for k in range(8):
                    w = rows_v[par, k * L + p]
                    a = plsc.bitcast(lax.shift_left(w, 16), jnp.float32)
                    b = plsc.bitcast(w & msk, jnp.float32)
                    lo.append(a * wb[k])
                    hi.append(b * wb[k])
                for h, t in ((0, lo), (1, hi)):
                    s01, s23 = t[0] + t[1], t[2] + t[3]
                    s45, s67 = t[4] + t[5], t[6] + t[7]
                    out_v[par, p, pl.ds(h * L, L)] = (s01 + s23) + (s45 + s67)

        def fire_out(g, par):
            pltpu.async_copy(out_v.at[par],
                             out_hbm.at[pl.ds(wbase + g * L, L)], sem_o)

        def drain_out(g, par):
            pltpu.make_async_copy(out_v.at[par],
                                  out_hbm.at[pl.ds(wbase + g * L, L)],
                                  sem_o).wait()

        a_phase(0, 0)
        fire_gather(0)

        def body(i, carry):
            for par in (0, 1):
                g = i * 2 + par
                nxt = 1 - par

                @pl.when(g + 1 < NG)
                def _prefetch():
                    a_phase(g + 1, nxt)
                    fire_gather(nxt)

                wait_gather(par)

                @pl.when(g >= 2)
                def _drain():
                    drain_out(g - 2, par)

                b_phase(g, par)
                fire_out(g, par)
            return carry

        lax.fori_loop(0, NG // 2, body, 0)
        drain_out(NG - 2, 0)
        drain_out(NG - 1, 1)

    return grid_sample_sc


def kernel(xc, voxel, offset, scale, ratio, ratio_dim):
    shape = xc.shape
    C, D, H, W = voxel.shape
    M = xc.size // 3

    table = _build_transpose(D * H * W, C)(voxel.reshape(C, D * H * W))

    sizes = jnp.array([W - 1, H - 1, D - 1], jnp.float32)
    r_mult = jnp.where(jnp.arange(3) == ratio_dim, ratio, 1.0)
    a = 0.5 * sizes * r_mult / scale
    b = 0.5 * sizes * (1.0 - r_mult * offset / scale)
    par = jnp.concatenate(
        [jnp.stack([a[0], b[0], a[1], b[1], a[2], b[2]]),
         jnp.zeros((2,), jnp.float32)])
    par16 = jnp.broadcast_to(par[:, None], (8, L))

    out = _build_sc_call(M, C, D, H, W)(xc.reshape(-1), table, par16)
    return out.reshape(shape[:-1] + (C,))


# speedup vs baseline: 1.0749x; 1.0749x over previous
import functools

import jax
import jax.numpy as jnp
from jax import lax
from jax.experimental import pallas as pl
from jax.experimental.pallas import tpu as pltpu
from jax.experimental.pallas import tpu_sc as plsc

L = 16


def _iota():
    return lax.broadcasted_iota(jnp.int32, (L,), 0)


_SPLAT_DN = lax.GatherDimensionNumbers(
    offset_dims=(), collapsed_slice_dims=(0,), start_index_map=(0,))


def _vsplat(v, p):
    idx = jnp.full((L, 1), p, jnp.int32)
    return lax.gather(v, idx, _SPLAT_DN, (1,),
                      mode=lax.GatherScatterMode.PROMISE_IN_BOUNDS)


@functools.cache
def _build_transpose(NCELL, C):
    info = plsc.get_sparse_core_info()
    NC, NS = info.num_cores, info.num_subcores
    NW = NC * NS
    CW = NCELL // NW
    BLK = 768
    NB = CW // BLK
    P = BLK + 1
    assert NCELL % NW == 0 and CW % (2 * BLK) == 0 and C == 2 * L

    mesh = plsc.VectorSubcoreMesh(core_axis_name="c", subcore_axis_name="s")

    @functools.partial(
        pl.kernel,
        out_type=jax.ShapeDtypeStruct((NCELL, C), jnp.float32),
        mesh=mesh,
        compiler_params=pltpu.CompilerParams(needs_layout_passes=False,
                                             use_tc_tiling_on_sc=False),
        scratch_types=[
            pltpu.VMEM((2, C, P), jnp.float32),
            pltpu.VMEM((2, BLK, C), jnp.float32),
            pltpu.SemaphoreType.DMA,
            pltpu.SemaphoreType.DMA,
        ],
    )
    def transpose_sc(vox_hbm, tab_hbm, stage_v, ob_v, sem_i, sem_o):
        wid = lax.axis_index("s") * NC + lax.axis_index("c")
        wbase = wid * CW
        lane = _iota()

        def fire_stage(b, par):
            pltpu.async_copy(vox_hbm.at[:, pl.ds(wbase + b * BLK, BLK)],
                             stage_v.at[par, :, pl.ds(0, BLK)], sem_i)

        def wait_stage(b, par):
            pltpu.make_async_copy(vox_hbm.at[:, pl.ds(wbase + b * BLK, BLK)],
                                  stage_v.at[par, :, pl.ds(0, BLK)],
                                  sem_i).wait()

        def fire_out(b, par):
            pltpu.async_copy(ob_v.at[par],
                             tab_hbm.at[pl.ds(wbase + b * BLK, BLK)], sem_o)

        def drain_out(b, par):
            pltpu.make_async_copy(ob_v.at[par],
                                  tab_hbm.at[pl.ds(wbase + b * BLK, BLK)],
                                  sem_o).wait()

        fire_stage(0, 0)
        a_lo = lane * P
        a_hi = (lane + L) * P

        def body(i, carry):
            for par in (0, 1):
                b = i * 2 + par
                nxt = 1 - par

                @pl.when(b + 1 < NB)
                def _prefetch():
                    fire_stage(b + 1, nxt)

                wait_stage(b, par)

                @pl.when(b >= 2)
                def _drain():
                    drain_out(b - 2, par)

                sp = stage_v.at[par]
                op = ob_v.at[par]

                def jbody(j4, carry2):
                    for u in range(4):
                        j = j4 * 4 + u
                        v0 = plsc.load_gather(sp, [lane, jnp.full((L,), j, jnp.int32)])
                        v1 = plsc.load_gather(sp, [lane + L, jnp.full((L,), j, jnp.int32)])
                        plsc.store_scatter(op, [jnp.full((L,), j, jnp.int32), lane], v0)
                        plsc.store_scatter(op, [jnp.full((L,), j, jnp.int32), lane + L], v1)
                    return carry2

                lax.fori_loop(0, BLK // 4, jbody, 0)
                fire_out(b, par)
            return carry

        lax.fori_loop(0, NB // 2, body, 0)
        drain_out(NB - 2, 0)
        drain_out(NB - 1, 1)

    return transpose_sc


@functools.cache
def _build_sc_call(M, C, D, H, W):
    info = plsc.get_sparse_core_info()
    NC, NS = info.num_cores, info.num_subcores
    NW = NC * NS
    PW = M // NW
    NG = PW // L
    CH = C // L
    assert M % NW == 0 and PW % (2 * L) == 0 and C % L == 0

    mesh = plsc.VectorSubcoreMesh(core_axis_name="c", subcore_axis_name="s")

    @functools.partial(
        pl.kernel,
        out_type=jax.ShapeDtypeStruct((M, C), jnp.float32),
        mesh=mesh,
        compiler_params=pltpu.CompilerParams(needs_layout_passes=False,
                                             use_tc_tiling_on_sc=False),
        scratch_types=[
            pltpu.VMEM((PW * 3,), jnp.float32),
            pltpu.VMEM((2, 8 * L), jnp.int32),
            pltpu.VMEM((2, 8 * L, C), jnp.float32),
            pltpu.VMEM((2, 8, L), jnp.float32),
            pltpu.VMEM((2, L, C), jnp.float32),
            pltpu.VMEM((8, L), jnp.float32),
            pltpu.SemaphoreType.DMA,
            pltpu.SemaphoreType.DMA,
        ],
    )
    def grid_sample_sc(xc_hbm, table_hbm, par_hbm, out_hbm,
                       coords_v, idx_v, rows_v, w_v, out_v, par_v,
                       sem_g, sem_o):
        wid = lax.axis_index("s") * NC + lax.axis_index("c")
        wbase = wid * PW
        pltpu.sync_copy(par_hbm, par_v)
        pltpu.sync_copy(xc_hbm.at[pl.ds(wbase * 3, PW * 3)], coords_v)
        lane = _iota()

        def a_phase(g, par):
            a3 = (g * L + lane) * 3
            gx = plsc.load_gather(coords_v, [a3])
            gy = plsc.load_gather(coords_v, [a3 + 1])
            gz = plsc.load_gather(coords_v, [a3 + 2])

            def axis(gc, arow, brow, size):
                i = jnp.minimum(jnp.maximum(par_v[arow] * gc + par_v[brow],
                                            0.0), float(size - 1))
                i0 = i.astype(jnp.int32)
                w1 = i - i0.astype(jnp.float32)
                i1 = jnp.minimum(i0 + 1, size - 1)
                return i0, i1, w1

            x0, x1, wx = axis(gx, 0, 1, W)
            y0, y1, wy = axis(gy, 2, 3, H)
            z0, z1, wz = axis(gz, 4, 5, D)

            zy = [z0 * (H * W) + y0 * W, z0 * (H * W) + y1 * W,
                  z1 * (H * W) + y0 * W, z1 * (H * W) + y1 * W]
            wzy = [(1.0 - wz) * (1.0 - wy), (1.0 - wz) * wy,
                   wz * (1.0 - wy), wz * wy]
            for k in range(8):
                r = zy[k >> 1] + (x0 if k & 1 == 0 else x1)
                wk = wzy[k >> 1] * ((1.0 - wx) if k & 1 == 0 else wx)
                idx_v[par, pl.ds(k * L, L)] = r
                w_v[par, k] = wk

        def fire_gather(par):
            pltpu.async_copy(table_hbm.at[idx_v.at[par]], rows_v.at[par],
                             sem_g)

        def wait_gather(par):
            pltpu.make_async_copy(table_hbm.at[idx_v.at[par]],
                                  rows_v.at[par], sem_g).wait()

        def b_phase(g, par):
            wv = [w_v[par, k] for k in range(8)]
            for p in range(L):
                wb = [_vsplat(wv[k], p) for k in range(8)]
                for h in range(CH):
                    cs = pl.ds(h * L, L)
                    t = [rows_v[par, k * L + p, cs] * wb[k] for k in range(8)]
                    s01, s23 = t[0] + t[1], t[2] + t[3]
                    s45, s67 = t[4] + t[5], t[6] + t[7]
                    out_v[par, p, cs] = (s01 + s23) + (s45 + s67)

        def fire_out(g, par):
            pltpu.async_copy(out_v.at[par],
                             out_hbm.at[pl.ds(wbase + g * L, L)], sem_o)

        def drain_out(g, par):
            pltpu.make_async_copy(out_v.at[par],
                                  out_hbm.at[pl.ds(wbase + g * L, L)],
                                  sem_o).wait()

        a_phase(0, 0)
        fire_gather(0)

        def body(i, carry):
            for par in (0, 1):
                g = i * 2 + par
                nxt = 1 - par

                @pl.when(g + 1 < NG)
                def _prefetch():
                    a_phase(g + 1, nxt)
                    fire_gather(nxt)

                wait_gather(par)

                @pl.when(g >= 2)
                def _drain():
                    drain_out(g - 2, par)

                b_phase(g, par)
                fire_out(g, par)
            return carry

        lax.fori_loop(0, NG // 2, body, 0)
        drain_out(NG - 2, 0)
        drain_out(NG - 1, 1)

    return grid_sample_sc


def kernel(xc, voxel, offset, scale, ratio, ratio_dim):
    shape = xc.shape
    C, D, H, W = voxel.shape
    M = xc.size // 3

    table = _build_transpose(D * H * W, C)(voxel.reshape(C, D * H * W))

    sizes = jnp.array([W - 1, H - 1, D - 1], jnp.float32)
    r_mult = jnp.where(jnp.arange(3) == ratio_dim, ratio, 1.0)
    a = 0.5 * sizes * r_mult / scale
    b = 0.5 * sizes * (1.0 - r_mult * offset / scale)
    par = jnp.concatenate(
        [jnp.stack([a[0], b[0], a[1], b[1], a[2], b[2]]),
         jnp.zeros((2,), jnp.float32)])
    par16 = jnp.broadcast_to(par[:, None], (8, L))

    out = _build_sc_call(M, C, D, H, W)(xc.reshape(-1), table, par16)
    return out.reshape(shape[:-1] + (C,))


# speedup vs baseline: 1.1016x; 1.0248x over previous
import functools

import jax
import jax.numpy as jnp
from jax import lax
from jax.experimental import pallas as pl
from jax.experimental.pallas import tpu as pltpu
from jax.experimental.pallas import tpu_sc as plsc

L = 16


def _iota():
    return lax.broadcasted_iota(jnp.int32, (L,), 0)


_SPLAT_DN = lax.GatherDimensionNumbers(
    offset_dims=(), collapsed_slice_dims=(0,), start_index_map=(0,))


def _vsplat(v, p):
    idx = jnp.full((L, 1), p, jnp.int32)
    return lax.gather(v, idx, _SPLAT_DN, (1,),
                      mode=lax.GatherScatterMode.PROMISE_IN_BOUNDS)


@functools.cache
def _build_transpose(NCELL, C):
    info = plsc.get_sparse_core_info()
    NC, NS = info.num_cores, info.num_subcores
    NW = NC * NS
    CW = NCELL // NW
    BLK = 768
    NB = CW // BLK
    P = BLK + 1
    assert NCELL % NW == 0 and CW % (2 * BLK) == 0 and C == 2 * L

    mesh = plsc.VectorSubcoreMesh(core_axis_name="c", subcore_axis_name="s")

    @functools.partial(
        pl.kernel,
        out_type=jax.ShapeDtypeStruct((NCELL, C), jnp.float32),
        mesh=mesh,
        compiler_params=pltpu.CompilerParams(needs_layout_passes=False,
                                             use_tc_tiling_on_sc=False),
        scratch_types=[
            pltpu.VMEM((2, C, P), jnp.float32),
            pltpu.VMEM((2, BLK, C), jnp.float32),
            pltpu.SemaphoreType.DMA,
            pltpu.SemaphoreType.DMA,
        ],
    )
    def transpose_sc(vox_hbm, tab_hbm, stage_v, ob_v, sem_i, sem_o):
        wid = lax.axis_index("s") * NC + lax.axis_index("c")
        wbase = wid * CW
        lane = _iota()

        def fire_stage(b, par):
            pltpu.async_copy(vox_hbm.at[:, pl.ds(wbase + b * BLK, BLK)],
                             stage_v.at[par, :, pl.ds(0, BLK)], sem_i)

        def wait_stage(b, par):
            pltpu.make_async_copy(vox_hbm.at[:, pl.ds(wbase + b * BLK, BLK)],
                                  stage_v.at[par, :, pl.ds(0, BLK)],
                                  sem_i).wait()

        def fire_out(b, par):
            pltpu.async_copy(ob_v.at[par],
                             tab_hbm.at[pl.ds(wbase + b * BLK, BLK)], sem_o)

        def drain_out(b, par):
            pltpu.make_async_copy(ob_v.at[par],
                                  tab_hbm.at[pl.ds(wbase + b * BLK, BLK)],
                                  sem_o).wait()

        fire_stage(0, 0)
        a_lo = lane * P
        a_hi = (lane + L) * P

        def body(i, carry):
            for par in (0, 1):
                b = i * 2 + par
                nxt = 1 - par

                @pl.when(b + 1 < NB)
                def _prefetch():
                    fire_stage(b + 1, nxt)

                wait_stage(b, par)

                @pl.when(b >= 2)
                def _drain():
                    drain_out(b - 2, par)

                sp = stage_v.at[par]
                op = ob_v.at[par]

                def jbody(j4, carry2):
                    for u in range(4):
                        j = j4 * 4 + u
                        v0 = plsc.load_gather(sp, [lane, jnp.full((L,), j, jnp.int32)])
                        v1 = plsc.load_gather(sp, [lane + L, jnp.full((L,), j, jnp.int32)])
                        plsc.store_scatter(op, [jnp.full((L,), j, jnp.int32), lane], v0)
                        plsc.store_scatter(op, [jnp.full((L,), j, jnp.int32), lane + L], v1)
                    return carry2

                lax.fori_loop(0, BLK // 4, jbody, 0)
                fire_out(b, par)
            return carry

        lax.fori_loop(0, NB // 2, body, 0)
        drain_out(NB - 2, 0)
        drain_out(NB - 1, 1)

    return transpose_sc


@functools.cache
def _build_sc_call(M, C, D, H, W):
    info = plsc.get_sparse_core_info()
    NC, NS = info.num_cores, info.num_subcores
    NW = NC * NS
    PW = M // NW
    NG = PW // L
    CH = C // L
    assert M % NW == 0 and PW % (2 * L) == 0 and C % L == 0

    mesh = plsc.VectorSubcoreMesh(core_axis_name="c", subcore_axis_name="s")

    @functools.partial(
        pl.kernel,
        out_type=jax.ShapeDtypeStruct((M, C), jnp.float32),
        mesh=mesh,
        compiler_params=pltpu.CompilerParams(needs_layout_passes=False,
                                             use_tc_tiling_on_sc=False),
        scratch_types=[
            pltpu.VMEM((PW * 3,), jnp.float32),
            pltpu.VMEM((4, 8 * L), jnp.int32),
            pltpu.VMEM((4, 8 * L, C), jnp.float32),
            pltpu.VMEM((4, 8, L), jnp.float32),
            pltpu.VMEM((4, L, C), jnp.float32),
            pltpu.VMEM((8, L), jnp.float32),
            pltpu.SemaphoreType.DMA,
            pltpu.SemaphoreType.DMA,
        ],
    )
    def grid_sample_sc(xc_hbm, table_hbm, par_hbm, out_hbm,
                       coords_v, idx_v, rows_v, w_v, out_v, par_v,
                       sem_g, sem_o):
        wid = lax.axis_index("s") * NC + lax.axis_index("c")
        wbase = wid * PW
        pltpu.sync_copy(par_hbm, par_v)
        pltpu.sync_copy(xc_hbm.at[pl.ds(wbase * 3, PW * 3)], coords_v)
        lane = _iota()

        def a_phase(g, par):
            a3 = (g * L + lane) * 3
            gx = plsc.load_gather(coords_v, [a3])
            gy = plsc.load_gather(coords_v, [a3 + 1])
            gz = plsc.load_gather(coords_v, [a3 + 2])

            def axis(gc, arow, brow, size):
                i = jnp.minimum(jnp.maximum(par_v[arow] * gc + par_v[brow],
                                            0.0), float(size - 1))
                i0 = i.astype(jnp.int32)
                w1 = i - i0.astype(jnp.float32)
                i1 = jnp.minimum(i0 + 1, size - 1)
                return i0, i1, w1

            x0, x1, wx = axis(gx, 0, 1, W)
            y0, y1, wy = axis(gy, 2, 3, H)
            z0, z1, wz = axis(gz, 4, 5, D)

            zy = [z0 * (H * W) + y0 * W, z0 * (H * W) + y1 * W,
                  z1 * (H * W) + y0 * W, z1 * (H * W) + y1 * W]
            wzy = [(1.0 - wz) * (1.0 - wy), (1.0 - wz) * wy,
                   wz * (1.0 - wy), wz * wy]
            for k in range(8):
                r = zy[k >> 1] + (x0 if k & 1 == 0 else x1)
                wk = wzy[k >> 1] * ((1.0 - wx) if k & 1 == 0 else wx)
                idx_v[par, pl.ds(k * L, L)] = r
                w_v[par, k] = wk

        def fire_gather(par):
            pltpu.async_copy(table_hbm.at[idx_v.at[par]], rows_v.at[par],
                             sem_g)

        def wait_gather(par):
            pltpu.make_async_copy(table_hbm.at[idx_v.at[par]],
                                  rows_v.at[par], sem_g).wait()

        def b_phase(g, par):
            wv = [w_v[par, k] for k in range(8)]
            for p in range(L):
                wb = [_vsplat(wv[k], p) for k in range(8)]
                for h in range(CH):
                    cs = pl.ds(h * L, L)
                    t = [rows_v[par, k * L + p, cs] * wb[k] for k in range(8)]
                    s01, s23 = t[0] + t[1], t[2] + t[3]
                    s45, s67 = t[4] + t[5], t[6] + t[7]
                    out_v[par, p, cs] = (s01 + s23) + (s45 + s67)

        def fire_out(g, par):
            pltpu.async_copy(out_v.at[par],
                             out_hbm.at[pl.ds(wbase + g * L, L)], sem_o)

        def drain_out(g, par):
            pltpu.make_async_copy(out_v.at[par],
                                  out_hbm.at[pl.ds(wbase + g * L, L)],
                                  sem_o).wait()

        for g0 in range(3):
            a_phase(g0, g0)
            fire_gather(g0)

        def body(i, carry):
            for par in range(4):
                g = i * 4 + par
                nxt = (par + 3) % 4

                @pl.when(g + 3 < NG)
                def _prefetch():
                    a_phase(g + 3, nxt)
                    fire_gather(nxt)

                wait_gather(par)

                @pl.when(g >= 4)
                def _drain():
                    drain_out(g - 4, par)

                b_phase(g, par)
                fire_out(g, par)
            return carry

        lax.fori_loop(0, NG // 4, body, 0)
        for g0 in range(NG - 4, NG):
            drain_out(g0, g0 % 4)

    return grid_sample_sc


def kernel(xc, voxel, offset, scale, ratio, ratio_dim):
    shape = xc.shape
    C, D, H, W = voxel.shape
    M = xc.size // 3

    table = _build_transpose(D * H * W, C)(voxel.reshape(C, D * H * W))

    sizes = jnp.array([W - 1, H - 1, D - 1], jnp.float32)
    r_mult = jnp.where(jnp.arange(3) == ratio_dim, ratio, 1.0)
    a = 0.5 * sizes * r_mult / scale
    b = 0.5 * sizes * (1.0 - r_mult * offset / scale)
    par = jnp.concatenate(
        [jnp.stack([a[0], b[0], a[1], b[1], a[2], b[2]]),
         jnp.zeros((2,), jnp.float32)])
    par16 = jnp.broadcast_to(par[:, None], (8, L))

    out = _build_sc_call(M, C, D, H, W)(xc.reshape(-1), table, par16)
    return out.reshape(shape[:-1] + (C,))


# speedup vs baseline: 1.1567x; 1.0500x over previous
import functools

import jax
import jax.numpy as jnp
from jax import lax
from jax.experimental import pallas as pl
from jax.experimental.pallas import tpu as pltpu
from jax.experimental.pallas import tpu_sc as plsc

L = 16


def _iota():
    return lax.broadcasted_iota(jnp.int32, (L,), 0)


_SPLAT_DN = lax.GatherDimensionNumbers(
    offset_dims=(), collapsed_slice_dims=(0,), start_index_map=(0,))


def _vsplat(v, p):
    idx = jnp.full((L, 1), p, jnp.int32)
    return lax.gather(v, idx, _SPLAT_DN, (1,),
                      mode=lax.GatherScatterMode.PROMISE_IN_BOUNDS)


@functools.cache
def _build_transpose(NCELL, C):
    info = plsc.get_sparse_core_info()
    NC, NS = info.num_cores, info.num_subcores
    NW = NC * NS
    CW = NCELL // NW
    BLK = 768
    NB = CW // BLK
    P = BLK + 1
    assert NCELL % NW == 0 and CW % (2 * BLK) == 0 and C == 2 * L

    mesh = plsc.VectorSubcoreMesh(core_axis_name="c", subcore_axis_name="s")

    @functools.partial(
        pl.kernel,
        out_type=jax.ShapeDtypeStruct((NCELL, C // 2), jnp.int32),
        mesh=mesh,
        compiler_params=pltpu.CompilerParams(needs_layout_passes=False,
                                             use_tc_tiling_on_sc=False),
        scratch_types=[
            pltpu.VMEM((2, C, P), jnp.float32),
            pltpu.VMEM((2, BLK, C // 2), jnp.int32),
            pltpu.SemaphoreType.DMA,
            pltpu.SemaphoreType.DMA,
        ],
    )
    def transpose_sc(vox_hbm, tab_hbm, stage_v, ob_v, sem_i, sem_o):
        wid = lax.axis_index("s") * NC + lax.axis_index("c")
        wbase = wid * CW
        lane = _iota()

        def fire_stage(b, par):
            pltpu.async_copy(vox_hbm.at[:, pl.ds(wbase + b * BLK, BLK)],
                             stage_v.at[par, :, pl.ds(0, BLK)], sem_i)

        def wait_stage(b, par):
            pltpu.make_async_copy(vox_hbm.at[:, pl.ds(wbase + b * BLK, BLK)],
                                  stage_v.at[par, :, pl.ds(0, BLK)],
                                  sem_i).wait()

        def fire_out(b, par):
            pltpu.async_copy(ob_v.at[par],
                             tab_hbm.at[pl.ds(wbase + b * BLK, BLK)], sem_o)

        def drain_out(b, par):
            pltpu.make_async_copy(ob_v.at[par],
                                  tab_hbm.at[pl.ds(wbase + b * BLK, BLK)],
                                  sem_o).wait()

        fire_stage(0, 0)
        a_lo = lane * P
        a_hi = (lane + L) * P

        def body(i, carry):
            for par in (0, 1):
                b = i * 2 + par
                nxt = 1 - par

                @pl.when(b + 1 < NB)
                def _prefetch():
                    fire_stage(b + 1, nxt)

                wait_stage(b, par)

                @pl.when(b >= 2)
                def _drain():
                    drain_out(b - 2, par)

                sp = stage_v.at[par]
                op = ob_v.at[par]

                def jbody(j4, carry2):
                    for u in range(4):
                        j = j4 * 4 + u
                        jv = jnp.full((L,), j, jnp.int32)
                        v0 = plsc.load_gather(sp, [lane, jv])
                        v1 = plsc.load_gather(sp, [lane + L, jv])
                        pk = plsc.pack(v0, v1, format=plsc.PackFormat.INTERLEAVED)
                        plsc.store_scatter(op, [jv, lane],
                                           plsc.bitcast(pk, jnp.int32))
                    return carry2

                lax.fori_loop(0, BLK // 4, jbody, 0)
                fire_out(b, par)
            return carry

        lax.fori_loop(0, NB // 2, body, 0)
        drain_out(NB - 2, 0)
        drain_out(NB - 1, 1)

    return transpose_sc


@functools.cache
def _build_sc_call(M, C, D, H, W):
    info = plsc.get_sparse_core_info()
    NC, NS = info.num_cores, info.num_subcores
    NW = NC * NS
    PW = M // NW
    NG = PW // L
    CH = C // L
    assert M % NW == 0 and PW % (2 * L) == 0 and C % L == 0

    mesh = plsc.VectorSubcoreMesh(core_axis_name="c", subcore_axis_name="s")

    @functools.partial(
        pl.kernel,
        out_type=jax.ShapeDtypeStruct((M, C), jnp.float32),
        mesh=mesh,
        compiler_params=pltpu.CompilerParams(needs_layout_passes=False,
                                             use_tc_tiling_on_sc=False),
        scratch_types=[
            pltpu.VMEM((PW * 3,), jnp.float32),
            pltpu.VMEM((4, 8 * L), jnp.int32),
            pltpu.VMEM((4, 8 * L, C // 2), jnp.int32),
            pltpu.VMEM((4, 8, L), jnp.float32),
            pltpu.VMEM((4, L, C), jnp.float32),
            pltpu.VMEM((8, L), jnp.float32),
            pltpu.SemaphoreType.DMA,
            pltpu.SemaphoreType.DMA,
        ],
    )
    def grid_sample_sc(xc_hbm, table_hbm, par_hbm, out_hbm,
                       coords_v, idx_v, rows_v, w_v, out_v, par_v,
                       sem_g, sem_o):
        wid = lax.axis_index("s") * NC + lax.axis_index("c")
        wbase = wid * PW
        pltpu.sync_copy(par_hbm, par_v)
        pltpu.sync_copy(xc_hbm.at[pl.ds(wbase * 3, PW * 3)], coords_v)
        lane = _iota()

        def a_phase(g, par):
            a3 = (g * L + lane) * 3
            gx = plsc.load_gather(coords_v, [a3])
            gy = plsc.load_gather(coords_v, [a3 + 1])
            gz = plsc.load_gather(coords_v, [a3 + 2])

            def axis(gc, arow, brow, size):
                i = jnp.minimum(jnp.maximum(par_v[arow] * gc + par_v[brow],
                                            0.0), float(size - 1))
                i0 = i.astype(jnp.int32)
                w1 = i - i0.astype(jnp.float32)
                i1 = jnp.minimum(i0 + 1, size - 1)
                return i0, i1, w1

            x0, x1, wx = axis(gx, 0, 1, W)
            y0, y1, wy = axis(gy, 2, 3, H)
            z0, z1, wz = axis(gz, 4, 5, D)

            zy = [z0 * (H * W) + y0 * W, z0 * (H * W) + y1 * W,
                  z1 * (H * W) + y0 * W, z1 * (H * W) + y1 * W]
            wzy = [(1.0 - wz) * (1.0 - wy), (1.0 - wz) * wy,
                   wz * (1.0 - wy), wz * wy]
            for k in range(8):
                r = zy[k >> 1] + (x0 if k & 1 == 0 else x1)
                wk = wzy[k >> 1] * ((1.0 - wx) if k & 1 == 0 else wx)
                idx_v[par, pl.ds(k * L, L)] = r
                w_v[par, k] = wk

        def fire_gather(par):
            pltpu.async_copy(table_hbm.at[idx_v.at[par]], rows_v.at[par],
                             sem_g)

        def wait_gather(par):
            pltpu.make_async_copy(table_hbm.at[idx_v.at[par]],
                                  rows_v.at[par], sem_g).wait()

        def b_phase(g, par):
            wv = [w_v[par, k] for k in range(8)]
            msk = jnp.full((L,), -0x10000, jnp.int32)
            for p in range(L):
                wb = [_vsplat(wv[k], p) for k in range(8)]
                lo, hi = [], []
                for k in range(8):
                    w = rows_v[par, k * L + p]
                    lo.append(plsc.bitcast(lax.shift_left(w, 16),
                                           jnp.float32) * wb[k])
                    hi.append(plsc.bitcast(w & msk, jnp.float32) * wb[k])
                for h, t in ((0, lo), (1, hi)):
                    s01, s23 = t[0] + t[1], t[2] + t[3]
                    s45, s67 = t[4] + t[5], t[6] + t[7]
                    out_v[par, p, pl.ds(h * L, L)] = (s01 + s23) + (s45 + s67)

        def fire_out(g, par):
            pltpu.async_copy(out_v.at[par],
                             out_hbm.at[pl.ds(wbase + g * L, L)], sem_o)

        def drain_out(g, par):
            pltpu.make_async_copy(out_v.at[par],
                                  out_hbm.at[pl.ds(wbase + g * L, L)],
                                  sem_o).wait()

        for g0 in range(3):
            a_phase(g0, g0)
            fire_gather(g0)

        def body(i, carry):
            for par in range(4):
                g = i * 4 + par
                nxt = (par + 3) % 4

                @pl.when(g + 3 < NG)
                def _prefetch():
                    a_phase(g + 3, nxt)
                    fire_gather(nxt)

                wait_gather(par)

                @pl.when(g >= 4)
                def _drain():
                    drain_out(g - 4, par)

                b_phase(g, par)
                fire_out(g, par)
            return carry

        lax.fori_loop(0, NG // 4, body, 0)
        for g0 in range(NG - 4, NG):
            drain_out(g0, g0 % 4)

    return grid_sample_sc


def kernel(xc, voxel, offset, scale, ratio, ratio_dim):
    shape = xc.shape
    C, D, H, W = voxel.shape
    M = xc.size // 3

    table = _build_transpose(D * H * W, C)(voxel.reshape(C, D * H * W))

    sizes = jnp.array([W - 1, H - 1, D - 1], jnp.float32)
    r_mult = jnp.where(jnp.arange(3) == ratio_dim, ratio, 1.0)
    a = 0.5 * sizes * r_mult / scale
    b = 0.5 * sizes * (1.0 - r_mult * offset / scale)
    par = jnp.concatenate(
        [jnp.stack([a[0], b[0], a[1], b[1], a[2], b[2]]),
         jnp.zeros((2,), jnp.float32)])
    par16 = jnp.broadcast_to(par[:, None], (8, L))

    out = _build_sc_call(M, C, D, H, W)(xc.reshape(-1), table, par16)
    return out.reshape(shape[:-1] + (C,))
